# Initial kernel scaffold; baseline (speedup 1.0000x reference)
import jax
import jax.numpy as jnp
from jax.experimental import pallas as pl


def kernel(input_ids, emb_table, W, b):
    raise NotImplementedError("write your pallas kernel here")


# speedup vs baseline: 66.7490x; 66.7490x over previous
import functools

import jax
import jax.numpy as jnp
from jax import lax
from jax.experimental import pallas as pl
from jax.experimental.pallas import tpu as pltpu
from jax.experimental.pallas import tpu_sc as plsc

VOCAB = 30522
D = 768
B = 4096
L = 200

LANES = 16
NC = 2
NS = 16
NW = NC * NS
RB = B // NW
GROUPS = RB // LANES

VBLK = 1024
NVBLK = -(-VOCAB // VBLK)
VPAD = NVBLK * VBLK


def _matvec_body(emb_ref, wt_ref, out_ref):
    x = emb_ref[...]
    w = wt_ref[...]
    out_ref[...] = jnp.sum(x * w, axis=1)


def _matvec(emb, wt):
    return pl.pallas_call(
        _matvec_body,
        grid=(NVBLK,),
        in_specs=[
            pl.BlockSpec((VBLK, D), lambda i: (i, 0)),
            pl.BlockSpec((1, D), lambda i: (0, 0)),
        ],
        out_specs=pl.BlockSpec((VBLK,), lambda i: (i,)),
        out_shape=jax.ShapeDtypeStruct((VPAD,), jnp.float32),
    )(emb, wt)


@functools.partial(
    pl.kernel,
    out_type=jax.ShapeDtypeStruct((B,), jnp.float32),
    mesh=plsc.VectorSubcoreMesh(core_axis_name="c", subcore_axis_name="s"),
    compiler_params=pltpu.CompilerParams(needs_layout_passes=False),
    scratch_types=[
        pltpu.VMEM((L, RB), jnp.int32),
        pltpu.VMEM((VPAD,), jnp.float32),
        pltpu.VMEM((LANES,), jnp.float32),
        pltpu.VMEM((RB,), jnp.float32),
    ],
)
def _sc_gather_mean(ids_hbm, v_hbm, b_hbm, out_hbm, ids_v, v_v, b_v, out_v):
    wid = lax.axis_index("s") * NC + lax.axis_index("c")
    base = wid * RB
    pltpu.sync_copy(v_hbm, v_v)
    pltpu.sync_copy(ids_hbm.at[:, pl.ds(base, RB)], ids_v)
    pltpu.sync_copy(b_hbm, b_v)
    inv_l = jnp.float32(1.0 / L)
    for g in range(GROUPS):
        def body(l, acc, g=g):
            idx = ids_v[l, pl.ds(g * LANES, LANES)]
            return acc + plsc.load_gather(v_v, [idx])
        acc = lax.fori_loop(0, L, body, jnp.zeros((LANES,), jnp.float32))
        out_v[pl.ds(g * LANES, LANES)] = acc * inv_l + b_v[...]
    pltpu.sync_copy(out_v, out_hbm.at[pl.ds(base, RB)])


def kernel(input_ids, emb_table, W, b):
    wt = W.reshape(1, D).astype(jnp.float32)
    v = _matvec(emb_table, wt)
    ids_t = input_ids.astype(jnp.int32).T
    b16 = jnp.broadcast_to(b.astype(jnp.float32), (LANES,))
    out = _sc_gather_mean(ids_t, v, b16)
    return out.reshape(B, 1)
